# baseline (speedup 1.0000x reference)
import functools

import jax
import jax.numpy as jnp
from jax import lax
from jax.experimental import pallas as pl
from jax.experimental.pallas import tpu as pltpu
from jax.experimental.pallas import tpu_sc as plsc

BATCH = 16384
D = 32
PACK = 128 // D
NUM_CORES = 2
NUM_SUBCORES = 16
NW = NUM_CORES * NUM_SUBCORES
B_PER_W = BATCH // NW
P_PER_W = B_PER_W // PACK
CHUNK = 128
NCHUNK = B_PER_W // CHUNK
LANES = 16


def _body(labels_hbm, mu_hbm, sigma_hbm, z_hbm, out_hbm,
          idx_v, ridx_v, z_v, mu_g, sigma_g, sem):
    wid = lax.axis_index("s") * NUM_CORES + lax.axis_index("c")
    base_lab = wid * B_PER_W
    base_pack = wid * P_PER_W

    pltpu.sync_copy(labels_hbm.at[pl.ds(base_lab, B_PER_W)], idx_v)

    def prep(v, _):
        sl = pl.ds(v * LANES, LANES)
        ridx_v[sl] = lax.shift_right_logical(idx_v[sl], 2)
        return 0

    lax.fori_loop(0, B_PER_W // LANES, prep, 0, unroll=4)

    pltpu.sync_copy(z_hbm.at[pl.ds(base_pack, P_PER_W)], z_v)

    for c in range(NCHUNK):
        csl = pl.ds(c * CHUNK, CHUNK)
        cp0 = pltpu.async_copy(mu_hbm.at[ridx_v.at[csl]], mu_g, sem)
        cp1 = pltpu.async_copy(sigma_hbm.at[ridx_v.at[csl]], sigma_g, sem)
        cp0.wait()
        cp1.wait()

        def row(i, _):
            j = c * (CHUNK // PACK) + lax.div(i, PACK)
            lab = idx_v[pl.ds(c * CHUNK + i, 1)][0]
            q = lax.shift_left(jnp.bitwise_and(lab, 3), 5)
            zoff = lax.rem(i, PACK) * D
            for hh in range(D // LANES):
                zsl = pl.ds(zoff + hh * LANES, LANES)
                gsl = pl.ds(q + hh * LANES, LANES)
                s = sigma_g[i, gsl]
                z_v[j, zsl] = z_v[j, zsl] * jnp.exp(0.5 * s) + mu_g[i, gsl]
            return 0

        lax.fori_loop(0, CHUNK, row, 0, unroll=4)

    pltpu.sync_copy(z_v, out_hbm.at[pl.ds(base_pack, P_PER_W)])


@jax.jit
def kernel(labels, mu_table, sigma_table, z):
    vocab = mu_table.shape[0]
    mu_p = mu_table.reshape(vocab // PACK, PACK * D)
    sigma_p = sigma_table.reshape(vocab // PACK, PACK * D)
    z_p = z.reshape(BATCH // PACK, PACK * D)
    mesh = plsc.VectorSubcoreMesh(core_axis_name="c", subcore_axis_name="s")
    k = functools.partial(
        pl.kernel,
        mesh=mesh,
        out_type=jax.ShapeDtypeStruct((BATCH // PACK, PACK * D), jnp.float32),
        scratch_types=[
            pltpu.VMEM((B_PER_W,), jnp.int32),
            pltpu.VMEM((B_PER_W,), jnp.int32),
            pltpu.VMEM((P_PER_W, PACK * D), jnp.float32),
            pltpu.VMEM((CHUNK, PACK * D), jnp.float32),
            pltpu.VMEM((CHUNK, PACK * D), jnp.float32),
            pltpu.SemaphoreType.DMA,
        ],
        compiler_params=pltpu.CompilerParams(use_tc_tiling_on_sc=True),
    )(_body)
    out_p = k(labels.astype(jnp.int32), mu_p, sigma_p, z_p)
    return out_p.reshape(BATCH, D)
